# baseline (speedup 1.0000x reference)
import functools

import jax
import jax.numpy as jnp
from jax import lax
from jax.experimental import pallas as pl
from jax.experimental.pallas import tpu as pltpu
from jax.experimental.pallas import tpu_sc as plsc

_H = 640
_HW = _H * _H
_B = 8
_N = _B * _HW
_RB = 160
_S = _H // _RB
_EPS = 1e-6


def _idx_body(b0, d0_ref, d1_ref, idx_ref):
    i = pl.program_id(0) + b0
    s = pl.program_id(1)
    d0 = d0_ref[0, 0]
    d1 = d1_ref[0, 0]
    row0 = (s * _RB).astype(jnp.float32)
    row = lax.broadcasted_iota(jnp.int32, (_RB, _H), 0).astype(jnp.float32) + row0
    col = lax.broadcasted_iota(jnp.int32, (_RB, _H), 1).astype(jnp.float32)
    offc = jnp.clip((col + 10.0 * d0).astype(jnp.int32), 0, _H - 1)
    offr = jnp.clip((row + 10.0 * d1).astype(jnp.int32), 0, _H - 1)
    idx_ref[0] = i * _HW + offr * _H + offc


def _make_indices(maps, b0, nb):
    return pl.pallas_call(
        functools.partial(_idx_body, b0),
        grid=(nb, _S),
        in_specs=[
            pl.BlockSpec((1, 1, _RB, _H), lambda i, s: (b0 + i, 1, s, 0)),
            pl.BlockSpec((1, 1, _RB, _H), lambda i, s: (b0 + i, 2, s, 0)),
        ],
        out_specs=pl.BlockSpec((1, _RB, _H), lambda i, s: (i, s, 0)),
        out_shape=jax.ShapeDtypeStruct((nb, _H, _H), jnp.int32),
    )(maps, maps)


def _sc_gather(idx_flat, table_flat):
    n = idx_flat.shape[0]
    info = plsc.get_sparse_core_info()
    nc, ns = info.num_cores, info.num_subcores
    nw = nc * ns
    n_per_w = n // nw
    n_chunks = 4
    ch = n_per_w // n_chunks
    mesh = plsc.VectorSubcoreMesh(core_axis_name="c", subcore_axis_name="s")

    @functools.partial(
        pl.kernel,
        out_type=jax.ShapeDtypeStruct((n,), jnp.int32),
        mesh=mesh,
        scratch_types=[
            pltpu.VMEM((ch,), jnp.int32),
            pltpu.VMEM((ch,), jnp.int32),
            pltpu.VMEM((ch,), jnp.int32),
            pltpu.VMEM((ch,), jnp.int32),
            pltpu.SemaphoreType.DMA,
            pltpu.SemaphoreType.DMA,
            pltpu.SemaphoreType.DMA,
            pltpu.SemaphoreType.DMA,
            pltpu.SemaphoreType.DMA,
            pltpu.SemaphoreType.DMA,
        ],
    )
    def gather_kernel(idx_hbm, table_hbm, out_hbm, idx_v0, idx_v1,
                      rows_v0, rows_v1, si0, si1, sg0, sg1, so0, so1):
        wid = lax.axis_index("s") * nc + lax.axis_index("c")
        base = wid * n_per_w
        idx_bufs = (idx_v0, idx_v1)
        row_bufs = (rows_v0, rows_v1)
        sems_i = (si0, si1)
        sems_g = (sg0, sg1)
        sems_o = (so0, so1)

        def load(j):
            return pltpu.async_copy(idx_hbm.at[pl.ds(base + j * ch, ch)],
                                    idx_bufs[j & 1], sems_i[j & 1])

        def gather(j):
            return pltpu.async_copy(table_hbm.at[idx_bufs[j & 1]],
                                    row_bufs[j & 1], sems_g[j & 1])

        def store(j):
            return pltpu.async_copy(row_bufs[j & 1],
                                    out_hbm.at[pl.ds(base + j * ch, ch)],
                                    sems_o[j & 1])

        load(0).wait()
        ld1 = load(1)
        g = [gather(0)]
        ld1.wait()
        g.append(gather(1))
        st = [None, None, None, None]
        for j in range(n_chunks):
            g[j].wait()
            st[j] = store(j)
            if j + 2 < n_chunks:
                load(j + 2).wait()
                st[j].wait()
                g.append(gather(j + 2))
        st[n_chunks - 2].wait()
        st[n_chunks - 1].wait()

    return gather_kernel(idx_flat, table_flat)


def _dice_body(score_ref, gt_ref, tm_ref, out_ref, key_ref):
    score = score_ref[0, 0]
    gt_pos = gt_ref[0] > 0
    tm_pos = tm_ref[0] > 0

    pos_num = jnp.sum((gt_pos & tm_pos).astype(jnp.int32))
    neg_mask = jnp.logical_and(jnp.logical_not(gt_pos), tm_pos)
    neg_count = jnp.sum(neg_mask.astype(jnp.int32))
    neg_num = jnp.minimum(pos_num * 3, neg_count)
    fallback = jnp.logical_or(pos_num == 0, neg_num == 0)

    bits = lax.bitcast_convert_type(score, jnp.uint32)
    bits = jnp.where(bits == jnp.uint32(0x80000000), jnp.uint32(0), bits)
    sign = bits >= jnp.uint32(0x80000000)
    key_all = jnp.where(sign, ~bits, bits | jnp.uint32(0x80000000))
    key_ref[...] = jnp.where(neg_mask, key_all, jnp.uint32(0))

    def search_cond(carry):
        b, _, c_acc = carry
        return jnp.logical_and(b < 32, c_acc != neg_num)

    def search_step(carry):
        b, t, c_acc = carry
        bitv = lax.shift_left(jnp.uint32(1), jnp.uint32(31) - b.astype(jnp.uint32))
        cand = jnp.bitwise_or(t, bitv)
        cnt = jnp.sum((key_ref[...] >= cand).astype(jnp.int32))
        take = cnt >= neg_num
        return (b + 1,
                jnp.where(take, cand, t),
                jnp.where(take, cnt, c_acc))

    _, thr, _ = lax.while_loop(
        search_cond, search_step,
        (jnp.int32(0), jnp.uint32(0), jnp.int32(-1)))

    selected = jnp.logical_and(jnp.logical_or(key_all >= thr, gt_pos), tm_pos)
    m = jnp.where(fallback, tm_pos.astype(jnp.float32),
                  selected.astype(jnp.float32))

    sig = 1.0 / (1.0 + jnp.exp(-score))
    gtf = gt_pos.astype(jnp.float32)
    a = jnp.sum(sig * gtf * m)
    bsum = jnp.sum(sig * sig * m)
    csum = jnp.sum(gtf * m)
    dice = 1.0 - 2.0 * a / (bsum + csum + 0.002)
    out_ref[0, 0, :] = jnp.full((128,), dice, dtype=jnp.float32)


def _dice_loss(maps, gt_kernels, training_masks):
    return pl.pallas_call(
        _dice_body,
        grid=(_B,),
        in_specs=[
            pl.BlockSpec((1, 1, _H, _H), lambda i: (i, 0, 0, 0)),
            pl.BlockSpec((1, _H, _H), lambda i: (i, 0, 0)),
            pl.BlockSpec((1, _H, _H), lambda i: (i, 0, 0)),
        ],
        out_specs=pl.BlockSpec((1, 1, 128), lambda i: (i, 0, 0)),
        out_shape=jax.ShapeDtypeStruct((_B, 1, 128), jnp.float32),
        scratch_shapes=[pltpu.VMEM((_H, _H), jnp.uint32)],
    )(maps, gt_kernels, training_masks)


_RB2 = 160
_S2 = _H // _RB2
_BH = _B // 2


def _loc_body(d0_ref, d1_ref, g0_ref, g1_ref, gath_ref, gti_ref, tmd_ref,
              dice_ref, out_ref, acc_ref):
    s = pl.program_id(1)

    @pl.when(s == 0)
    def _():
        acc_ref[0] = 0.0
        acc_ref[1] = 0.0

    stm = jnp.logical_and(gath_ref[0] != gti_ref[0], tmd_ref[0] > 0)
    stm_f = stm.astype(jnp.float32)

    def huber(d, g):
        diff = jnp.abs(d - g) * stm_f
        return jnp.where(diff < 0.1, 5.0 * diff * diff, diff - 0.05)

    num = jnp.sum(huber(d0_ref[0, 0], g0_ref[0, 0])
                  + huber(d1_ref[0, 0], g1_ref[0, 0]))
    den = jnp.sum(stm_f)
    acc_ref[0] += num
    acc_ref[1] += den

    @pl.when(s == _S2 - 1)
    def _():
        loc = 0.05 * acc_ref[0] / (acc_ref[1] + _EPS)
        out_ref[0, 0, :] = dice_ref[0, 0, :] + loc


def _final_loss_part(b0, nb, maps, gt_distances, gathered_part, gt_instances,
                     training_mask_distances, dice):
    return pl.pallas_call(
        _loc_body,
        grid=(nb, _S2),
        in_specs=[
            pl.BlockSpec((1, 1, _RB2, _H), lambda i, s: (b0 + i, 1, s, 0)),
            pl.BlockSpec((1, 1, _RB2, _H), lambda i, s: (b0 + i, 2, s, 0)),
            pl.BlockSpec((1, 1, _RB2, _H), lambda i, s: (b0 + i, 0, s, 0)),
            pl.BlockSpec((1, 1, _RB2, _H), lambda i, s: (b0 + i, 1, s, 0)),
            pl.BlockSpec((1, _RB2, _H), lambda i, s: (i, s, 0)),
            pl.BlockSpec((1, _RB2, _H), lambda i, s: (b0 + i, s, 0)),
            pl.BlockSpec((1, _RB2, _H), lambda i, s: (b0 + i, s, 0)),
            pl.BlockSpec((1, 1, 128), lambda i, s: (b0 + i, 0, 0)),
        ],
        out_specs=pl.BlockSpec((1, 1, 128), lambda i, s: (i, 0, 0)),
        out_shape=jax.ShapeDtypeStruct((nb, 1, 128), jnp.float32),
        scratch_shapes=[pltpu.SMEM((2,), jnp.float32)],
    )(maps, maps, gt_distances, gt_distances, gathered_part, gt_instances,
      training_mask_distances, dice)


def kernel(maps, imgs, gt_kernels, training_masks, gt_instances,
           gt_kernel_instances, training_mask_distances, gt_distances):
    del imgs
    table = gt_kernel_instances.reshape(-1)
    parts = ((0, 1), (1, 3), (4, 3), (7, 1))
    gaths = []
    for b0, nb in parts:
        idx = _make_indices(maps, b0, nb)
        gaths.append(_sc_gather(idx.reshape(-1), table))
    dice = _dice_loss(maps, gt_kernels, training_masks)
    outs = []
    for k, (b0, nb) in enumerate(parts):
        outs.append(_final_loss_part(b0, nb, maps, gt_distances,
                                     gaths[k].reshape(nb, _H, _H),
                                     gt_instances, training_mask_distances,
                                     dice)[:, 0, 0])
    return jnp.concatenate(outs, axis=0)


# speedup vs baseline: 1.0283x; 1.0283x over previous
import functools

import jax
import jax.numpy as jnp
from jax import lax
from jax.experimental import pallas as pl
from jax.experimental.pallas import tpu as pltpu
from jax.experimental.pallas import tpu_sc as plsc

_H = 640
_HW = _H * _H
_B = 8
_N = _B * _HW
_RB = 160
_S = _H // _RB
_EPS = 1e-6


def _idx_body(b0, d0_ref, d1_ref, idx_ref):
    i = pl.program_id(0) + b0
    s = pl.program_id(1)
    d0 = d0_ref[0, 0]
    d1 = d1_ref[0, 0]
    row0 = (s * _RB).astype(jnp.float32)
    row = lax.broadcasted_iota(jnp.int32, (_RB, _H), 0).astype(jnp.float32) + row0
    col = lax.broadcasted_iota(jnp.int32, (_RB, _H), 1).astype(jnp.float32)
    offc = jnp.clip((col + 10.0 * d0).astype(jnp.int32), 0, _H - 1)
    offr = jnp.clip((row + 10.0 * d1).astype(jnp.int32), 0, _H - 1)
    idx_ref[0] = i * _HW + offr * _H + offc


def _make_indices(maps, b0, nb):
    return pl.pallas_call(
        functools.partial(_idx_body, b0),
        grid=(nb, _S),
        in_specs=[
            pl.BlockSpec((1, 1, _RB, _H), lambda i, s: (b0 + i, 1, s, 0)),
            pl.BlockSpec((1, 1, _RB, _H), lambda i, s: (b0 + i, 2, s, 0)),
        ],
        out_specs=pl.BlockSpec((1, _RB, _H), lambda i, s: (i, s, 0)),
        out_shape=jax.ShapeDtypeStruct((nb, _H, _H), jnp.int32),
    )(maps, maps)


def _sc_gather(idx_flat, table_flat):
    n = idx_flat.shape[0]
    info = plsc.get_sparse_core_info()
    nc, ns = info.num_cores, info.num_subcores
    nw = nc * ns
    n_per_w = n // nw
    n_chunks = 4
    ch = n_per_w // n_chunks
    mesh = plsc.VectorSubcoreMesh(core_axis_name="c", subcore_axis_name="s")

    @functools.partial(
        pl.kernel,
        out_type=jax.ShapeDtypeStruct((n,), jnp.int32),
        mesh=mesh,
        scratch_types=[
            pltpu.VMEM((ch,), jnp.int32),
            pltpu.VMEM((ch,), jnp.int32),
            pltpu.VMEM((ch,), jnp.int32),
            pltpu.VMEM((ch,), jnp.int32),
            pltpu.SemaphoreType.DMA,
            pltpu.SemaphoreType.DMA,
            pltpu.SemaphoreType.DMA,
            pltpu.SemaphoreType.DMA,
            pltpu.SemaphoreType.DMA,
            pltpu.SemaphoreType.DMA,
        ],
    )
    def gather_kernel(idx_hbm, table_hbm, out_hbm, idx_v0, idx_v1,
                      rows_v0, rows_v1, si0, si1, sg0, sg1, so0, so1):
        wid = lax.axis_index("s") * nc + lax.axis_index("c")
        base = wid * n_per_w
        idx_bufs = (idx_v0, idx_v1)
        row_bufs = (rows_v0, rows_v1)
        sems_i = (si0, si1)
        sems_g = (sg0, sg1)
        sems_o = (so0, so1)

        def load(j):
            return pltpu.async_copy(idx_hbm.at[pl.ds(base + j * ch, ch)],
                                    idx_bufs[j & 1], sems_i[j & 1])

        def gather(j):
            return pltpu.async_copy(table_hbm.at[idx_bufs[j & 1]],
                                    row_bufs[j & 1], sems_g[j & 1])

        def store(j):
            return pltpu.async_copy(row_bufs[j & 1],
                                    out_hbm.at[pl.ds(base + j * ch, ch)],
                                    sems_o[j & 1])

        load(0).wait()
        ld1 = load(1)
        g = [gather(0)]
        ld1.wait()
        g.append(gather(1))
        st = [None, None, None, None]
        for j in range(n_chunks):
            g[j].wait()
            st[j] = store(j)
            if j + 2 < n_chunks:
                load(j + 2).wait()
                st[j].wait()
                g.append(gather(j + 2))
        st[n_chunks - 2].wait()
        st[n_chunks - 1].wait()

    return gather_kernel(idx_flat, table_flat)


def _dice_body(score_ref, gt_ref, tm_ref, out_ref, key_ref):
    score = score_ref[0, 0]
    gt_pos = gt_ref[0] > 0
    tm_pos = tm_ref[0] > 0

    pos_num = jnp.sum((gt_pos & tm_pos).astype(jnp.int32))
    neg_mask = jnp.logical_and(jnp.logical_not(gt_pos), tm_pos)
    neg_count = jnp.sum(neg_mask.astype(jnp.int32))
    neg_num = jnp.minimum(pos_num * 3, neg_count)
    fallback = jnp.logical_or(pos_num == 0, neg_num == 0)

    bits = lax.bitcast_convert_type(score, jnp.uint32)
    bits = jnp.where(bits == jnp.uint32(0x80000000), jnp.uint32(0), bits)
    sign = bits >= jnp.uint32(0x80000000)
    key_all = jnp.where(sign, ~bits, bits | jnp.uint32(0x80000000))
    key_ref[...] = jnp.where(neg_mask, key_all, jnp.uint32(0))

    def search_cond(carry):
        b, _, c_acc = carry
        return jnp.logical_and(b < 32, c_acc != neg_num)

    def search_step(carry):
        b, t, c_acc = carry
        bitv = lax.shift_left(jnp.uint32(1), jnp.uint32(31) - b.astype(jnp.uint32))
        cand = jnp.bitwise_or(t, bitv)
        cnt = jnp.sum((key_ref[...] >= cand).astype(jnp.int32))
        take = cnt >= neg_num
        return (b + 1,
                jnp.where(take, cand, t),
                jnp.where(take, cnt, c_acc))

    _, thr, _ = lax.while_loop(
        search_cond, search_step,
        (jnp.int32(0), jnp.uint32(0), jnp.int32(-1)))

    selected = jnp.logical_and(jnp.logical_or(key_all >= thr, gt_pos), tm_pos)
    m = jnp.where(fallback, tm_pos.astype(jnp.float32),
                  selected.astype(jnp.float32))

    sig = 1.0 / (1.0 + jnp.exp(-score))
    gtf = gt_pos.astype(jnp.float32)
    a = jnp.sum(sig * gtf * m)
    bsum = jnp.sum(sig * sig * m)
    csum = jnp.sum(gtf * m)
    dice = 1.0 - 2.0 * a / (bsum + csum + 0.002)
    out_ref[0, 0, :] = jnp.full((128,), dice, dtype=jnp.float32)


def _dice_loss(maps, gt_kernels, training_masks):
    return pl.pallas_call(
        _dice_body,
        grid=(_B,),
        in_specs=[
            pl.BlockSpec((1, 1, _H, _H), lambda i: (i, 0, 0, 0)),
            pl.BlockSpec((1, _H, _H), lambda i: (i, 0, 0)),
            pl.BlockSpec((1, _H, _H), lambda i: (i, 0, 0)),
        ],
        out_specs=pl.BlockSpec((1, 1, 128), lambda i: (i, 0, 0)),
        out_shape=jax.ShapeDtypeStruct((_B, 1, 128), jnp.float32),
        scratch_shapes=[pltpu.VMEM((_H, _H), jnp.uint32)],
    )(maps, gt_kernels, training_masks)


_RB2 = 160
_S2 = _H // _RB2
_BH = _B // 2


def _loc_body(d0_ref, d1_ref, g0_ref, g1_ref, gath_ref, gti_ref, tmd_ref,
              dice_ref, out_ref, acc_ref):
    s = pl.program_id(1)

    @pl.when(s == 0)
    def _():
        acc_ref[0] = 0.0
        acc_ref[1] = 0.0

    stm = jnp.logical_and(gath_ref[0] != gti_ref[0], tmd_ref[0] > 0)
    stm_f = stm.astype(jnp.float32)

    def huber(d, g):
        diff = jnp.abs(d - g) * stm_f
        return jnp.where(diff < 0.1, 5.0 * diff * diff, diff - 0.05)

    num = jnp.sum(huber(d0_ref[0, 0], g0_ref[0, 0])
                  + huber(d1_ref[0, 0], g1_ref[0, 0]))
    den = jnp.sum(stm_f)
    acc_ref[0] += num
    acc_ref[1] += den

    @pl.when(s == _S2 - 1)
    def _():
        loc = 0.05 * acc_ref[0] / (acc_ref[1] + _EPS)
        out_ref[0, 0, :] = dice_ref[0, 0, :] + loc


def _final_loss_part(b0, nb, maps, gt_distances, gathered_part, gt_instances,
                     training_mask_distances, dice):
    return pl.pallas_call(
        _loc_body,
        grid=(nb, _S2),
        in_specs=[
            pl.BlockSpec((1, 1, _RB2, _H), lambda i, s: (b0 + i, 1, s, 0)),
            pl.BlockSpec((1, 1, _RB2, _H), lambda i, s: (b0 + i, 2, s, 0)),
            pl.BlockSpec((1, 1, _RB2, _H), lambda i, s: (b0 + i, 0, s, 0)),
            pl.BlockSpec((1, 1, _RB2, _H), lambda i, s: (b0 + i, 1, s, 0)),
            pl.BlockSpec((1, _RB2, _H), lambda i, s: (i, s, 0)),
            pl.BlockSpec((1, _RB2, _H), lambda i, s: (b0 + i, s, 0)),
            pl.BlockSpec((1, _RB2, _H), lambda i, s: (b0 + i, s, 0)),
            pl.BlockSpec((1, 1, 128), lambda i, s: (b0 + i, 0, 0)),
        ],
        out_specs=pl.BlockSpec((1, 1, 128), lambda i, s: (i, 0, 0)),
        out_shape=jax.ShapeDtypeStruct((nb, 1, 128), jnp.float32),
        scratch_shapes=[pltpu.SMEM((2,), jnp.float32)],
    )(maps, maps, gt_distances, gt_distances, gathered_part, gt_instances,
      training_mask_distances, dice)


def kernel(maps, imgs, gt_kernels, training_masks, gt_instances,
           gt_kernel_instances, training_mask_distances, gt_distances):
    del imgs
    table = gt_kernel_instances.reshape(-1)
    parts = ((0, 4), (4, 4))
    gaths = []
    for b0, nb in parts:
        idx = _make_indices(maps, b0, nb)
        gaths.append(_sc_gather(idx.reshape(-1), table))
    dice = _dice_loss(maps, gt_kernels, training_masks)
    outs = []
    for k, (b0, nb) in enumerate(parts):
        outs.append(_final_loss_part(b0, nb, maps, gt_distances,
                                     gaths[k].reshape(nb, _H, _H),
                                     gt_instances, training_mask_distances,
                                     dice)[:, 0, 0])
    return jnp.concatenate(outs, axis=0)
